# baseline (speedup 1.0000x reference)
import functools

import jax
import jax.numpy as jnp
from jax import lax
from jax.experimental import pallas as pl
from jax.experimental.pallas import tpu as pltpu
from jax.experimental.pallas import tpu_sc as plsc

N = 10000
E = 320000
D = 128

NC = 2
NS = 16
NW = NC * NS

CH = 128
NCH = -(-E // (NW * CH))
C2 = 224
NG = -(-E // (NW * C2))
E_PAD = NW * NG * C2
SLAB_CH = 9
SLAB_E = SLAB_CH * C2
A0 = 4
A1 = 6
SLABS = NS * (A0 + A1)
RPT = -(-(N + 1) // (NS * 8)) * 8
NP = NS * RPT
DUMMY = N


def _zero_buf(buf, rows, cols):
  zeros = jnp.zeros((16,), jnp.float32)

  def body(r, carry):
    for k in range(cols // 16):
      buf[r, pl.ds(k * 16, 16)] = zeros
    return carry

  lax.fori_loop(0, rows, body, 0)


@functools.lru_cache(maxsize=None)
def _make_sc_kernels():
  _mesh = plsc.VectorSubcoreMesh(core_axis_name="c", subcore_axis_name="s")

  E_W = NG * C2

  def deg_body(src_hbm, out_hbm, src_v, histo):
    c = lax.axis_index("c")
    s = lax.axis_index("s")
    w = c * NS + s
    pltpu.sync_copy(src_hbm.at[w], src_v)
    zeros = jnp.zeros((16,), jnp.float32)

    def zrow(r, carry):
      histo[pl.ds(r * 16, 16)] = zeros
      return carry

    lax.fori_loop(0, NP // 16, zrow, 0)
    ones = jnp.full((16,), 1.0, jnp.float32)

    def body(j, carry):
      idx = src_v[pl.ds(j * 16, 16)]
      plsc.addupdate_scatter(histo, [idx], ones)
      return carry

    lax.fori_loop(0, E_W // 16, body, 0)
    pltpu.sync_copy(histo, out_hbm.at[c, s])

  deg_kernel = pl.kernel(
      deg_body,
      out_type=jax.ShapeDtypeStruct((NC, NS, NP), jnp.float32),
      mesh=_mesh,
      compiler_params=pltpu.CompilerParams(needs_layout_passes=False),
      scratch_types=[
          pltpu.VMEM((E_W,), jnp.int32),
          pltpu.VMEM((NP,), jnp.float32),
      ],
  )

  def agg_body(y_hbm, src_hbm, dstp_hbm, out_hbm, src_v, dstp_v,
               buf, acc_sh, g0):
    c = lax.axis_index("c")
    s = lax.axis_index("s")
    _zero_buf(buf, C2, D)
    base = s * RPT
    nfull = RPT // 128
    for j in range(nfull):
      pltpu.sync_copy(buf.at[pl.ds(0, 128)],
                      acc_sh.at[pl.ds(base + j * 128, 128)])
    rem = RPT - nfull * 128
    if rem:
      pltpu.sync_copy(buf.at[pl.ds(0, rem)],
                      acc_sh.at[pl.ds(base + nfull * 128, rem)])
    plsc.subcore_barrier()

    base_c = c * (NS * A0)
    a_c = jnp.where(c == 0, A0, A1)

    def slab_loop(k, carry):
      sid = base_c + k * NS + s
      pltpu.sync_copy(src_hbm.at[sid], src_v)
      pltpu.sync_copy(dstp_hbm.at[sid], dstp_v)

      def body(g, carry2):
        pltpu.async_copy(y_hbm.at[src_v.at[pl.ds(g * C2, C2)]], buf,
                         g0).wait()
        pltpu.sync_copy(buf, acc_sh.at[dstp_v.at[pl.ds(g * C2, C2)]],
                        add=True)
        return carry2

      lax.fori_loop(0, SLAB_CH, body, 0)
      return carry

    lax.fori_loop(0, a_c, slab_loop, 0)
    plsc.subcore_barrier()
    pltpu.sync_copy(acc_sh.at[pl.ds(base, RPT)], out_hbm.at[c, s])

  agg_kernel = pl.kernel(
      agg_body,
      out_type=jax.ShapeDtypeStruct((NC, NS, RPT, D), jnp.float32),
      mesh=_mesh,
      scratch_types=[
          pltpu.VMEM((SLAB_E,), jnp.int32),
          pltpu.VMEM((SLAB_E,), jnp.int32),
          pltpu.VMEM((C2, D), jnp.float32),
          pltpu.VMEM_SHARED((NP, D), jnp.float32),
          pltpu.SemaphoreType.DMA,
      ],
  )
  return deg_kernel, agg_kernel


R = 1000


def _tc_prep_body(degp_ref, x_ref, dis_ref, y_ref):
  p = degp_ref[...]
  deg = jnp.sum(p, axis=0)
  dis = jnp.where(deg > 0, lax.rsqrt(jnp.maximum(deg, 1e-12)), 0.0)
  disb = jnp.broadcast_to(dis, (R, D))
  dis_ref[...] = disb
  y_ref[...] = disb * x_ref[...]


def _tc_prep(degp, x):
  return pl.pallas_call(
      _tc_prep_body,
      grid=(N // R,),
      in_specs=[
          pl.BlockSpec((NW, R, 1), lambda i: (0, i, 0)),
          pl.BlockSpec((R, D), lambda i: (i, 0)),
      ],
      out_specs=[
          pl.BlockSpec((R, D), lambda i: (i, 0)),
          pl.BlockSpec((R, D), lambda i: (i, 0)),
      ],
      out_shape=[
          jax.ShapeDtypeStruct((N, D), jnp.float32),
          jax.ShapeDtypeStruct((N, D), jnp.float32),
      ],
  )(degp, x)


def _tc_layer_a_body(x_ref, s0_ref, s1_ref, dis_ref, w0_ref, w1_ref, ba_ref,
                     g_ref, b_ref, h2_ref, y2_ref):
  x = x_ref[...]
  dis = dis_ref[...]
  tx1 = -dis * (s0_ref[...] + s1_ref[...])
  h = (jnp.dot(x, w0_ref[...], preferred_element_type=jnp.float32)
       + jnp.dot(tx1, w1_ref[...], preferred_element_type=jnp.float32)
       + ba_ref[...])
  mu = jnp.mean(h, axis=-1, keepdims=True)
  var = jnp.mean((h - mu) * (h - mu), axis=-1, keepdims=True)
  hn = (h - mu) / jnp.sqrt(var + 1e-5) * g_ref[...] + b_ref[...]
  h2 = jnp.where(hn >= 0, hn, 0.01 * hn)
  h2_ref[...] = h2
  y2_ref[...] = dis * h2


def _tc_layer_a(x, s0, s1, dis, w0, w1, ba, gamma, beta):
  row = lambda i: (i, 0)
  full = lambda i: (0, 0)
  return pl.pallas_call(
      _tc_layer_a_body,
      grid=(N // R,),
      in_specs=[
          pl.BlockSpec((R, D), row),
          pl.BlockSpec((R, D), row),
          pl.BlockSpec((R, D), row),
          pl.BlockSpec((R, D), row),
          pl.BlockSpec((D, D), full),
          pl.BlockSpec((D, D), full),
          pl.BlockSpec((1, D), full),
          pl.BlockSpec((1, D), full),
          pl.BlockSpec((1, D), full),
      ],
      out_specs=[
          pl.BlockSpec((R, D), row),
          pl.BlockSpec((R, D), row),
      ],
      out_shape=[
          jax.ShapeDtypeStruct((N, D), jnp.float32),
          jax.ShapeDtypeStruct((N, D), jnp.float32),
      ],
  )(x, s0, s1, dis, w0, w1, ba.reshape(1, D), gamma.reshape(1, D),
    beta.reshape(1, D))


def _tc_layer_b_body(h_ref, s0_ref, s1_ref, dis_ref, w0_ref, w1_ref, bb_ref,
                     out_ref):
  h = h_ref[...]
  tx1 = -dis_ref[...] * (s0_ref[...] + s1_ref[...])
  out_ref[...] = (jnp.dot(h, w0_ref[...], preferred_element_type=jnp.float32)
                  + jnp.dot(tx1, w1_ref[...], preferred_element_type=jnp.float32)
                  + bb_ref[...])


def _tc_layer_b(h, s0, s1, dis, w0, w1, bb):
  row = lambda i: (i, 0)
  full = lambda i: (0, 0)
  return pl.pallas_call(
      _tc_layer_b_body,
      grid=(N // R,),
      in_specs=[
          pl.BlockSpec((R, D), row),
          pl.BlockSpec((R, D), row),
          pl.BlockSpec((R, D), row),
          pl.BlockSpec((R, D), row),
          pl.BlockSpec((D, D), full),
          pl.BlockSpec((D, D), full),
          pl.BlockSpec((1, D), full),
      ],
      out_specs=pl.BlockSpec((R, D), row),
      out_shape=jax.ShapeDtypeStruct((N, D), jnp.float32),
  )(h, s0, s1, dis, w0, w1, bb.reshape(1, D))


def kernel(x, edge_index, W0a, W1a, ba, gamma, beta, W0b, W1b, bb):
  src = edge_index[0]
  dst = edge_index[1]
  pad = E_PAD - E
  src_p = jnp.concatenate([src, jnp.zeros((pad,), jnp.int32)]).reshape(
      SLABS, SLAB_E)
  dst_p = jnp.concatenate([dst, jnp.full((pad,), DUMMY, jnp.int32)]).reshape(
      SLABS, SLAB_E)
  src_deg = jnp.concatenate([src, jnp.full((pad,), DUMMY, jnp.int32)]).reshape(
      NW, NG * C2)
  _sc_deg, _sc_agg = _make_sc_kernels()

  degp = _sc_deg(src_deg).reshape(NW, NP)[:, :N].reshape(NW, N, 1)
  dis, y = _tc_prep(degp, x)

  sp = _sc_agg(y, src_p, dst_p).reshape(NC, NP, D)
  h2, y2 = _tc_layer_a(x, sp[0, :N], sp[1, :N], dis, W0a, W1a, ba, gamma,
                       beta)

  sp2 = _sc_agg(y2, src_p, dst_p).reshape(NC, NP, D)
  return _tc_layer_b(h2, sp2[0, :N], sp2[1, :N], dis, W0b, W1b, bb)


# speedup vs baseline: 1.1277x; 1.1277x over previous
import functools

import jax
import jax.numpy as jnp
from jax import lax
from jax.experimental import pallas as pl
from jax.experimental.pallas import tpu as pltpu
from jax.experimental.pallas import tpu_sc as plsc

N = 10000
E = 320000
D = 128

NC = 2
NS = 16
NW = NC * NS

CH = 128
NCH = -(-E // (NW * CH))
C2 = 224
NG = -(-E // (NW * C2))
E_PAD = NW * NG * C2
SLAB_CH = 9
SLAB_E = SLAB_CH * C2
A0 = 6
A1 = 4
SLABS = NS * (A0 + A1)
RPT = -(-(N + 1) // (NS * 8)) * 8
NP = NS * RPT
DUMMY = N


def _zero_buf(buf, rows, cols):
  zeros = jnp.zeros((16,), jnp.float32)

  def body(r, carry):
    for k in range(cols // 16):
      buf[r, pl.ds(k * 16, 16)] = zeros
    return carry

  lax.fori_loop(0, rows, body, 0)


@functools.lru_cache(maxsize=None)
def _make_sc_kernels():
  _mesh = plsc.VectorSubcoreMesh(core_axis_name="c", subcore_axis_name="s")

  E_W = NG * C2

  def deg_body(src_hbm, out_hbm, src_v, histo):
    c = lax.axis_index("c")
    s = lax.axis_index("s")
    w = c * NS + s
    pltpu.sync_copy(src_hbm.at[w], src_v)
    zeros = jnp.zeros((16,), jnp.float32)

    def zrow(r, carry):
      histo[pl.ds(r * 16, 16)] = zeros
      return carry

    lax.fori_loop(0, NP // 16, zrow, 0)
    ones = jnp.full((16,), 1.0, jnp.float32)

    def body(j, carry):
      idx = src_v[pl.ds(j * 16, 16)]
      plsc.addupdate_scatter(histo, [idx], ones)
      return carry

    lax.fori_loop(0, E_W // 16, body, 0)
    pltpu.sync_copy(histo, out_hbm.at[c, s])

  deg_kernel = pl.kernel(
      deg_body,
      out_type=jax.ShapeDtypeStruct((NC, NS, NP), jnp.float32),
      mesh=_mesh,
      compiler_params=pltpu.CompilerParams(needs_layout_passes=False),
      scratch_types=[
          pltpu.VMEM((E_W,), jnp.int32),
          pltpu.VMEM((NP,), jnp.float32),
      ],
  )

  def agg_body(y_hbm, src_hbm, dstp_hbm, out_hbm, src_v, dstp_v,
               buf, acc_sh, g0):
    c = lax.axis_index("c")
    s = lax.axis_index("s")
    _zero_buf(buf, C2, D)
    base = s * RPT
    nfull = RPT // 128
    for j in range(nfull):
      pltpu.sync_copy(buf.at[pl.ds(0, 128)],
                      acc_sh.at[pl.ds(base + j * 128, 128)])
    rem = RPT - nfull * 128
    if rem:
      pltpu.sync_copy(buf.at[pl.ds(0, rem)],
                      acc_sh.at[pl.ds(base + nfull * 128, rem)])
    plsc.subcore_barrier()

    base_c = c * (NS * A0)
    a_c = jnp.where(c == 0, A0, A1)

    def slab_loop(k, carry):
      sid = base_c + k * NS + s
      pltpu.sync_copy(src_hbm.at[sid], src_v)
      pltpu.sync_copy(dstp_hbm.at[sid], dstp_v)

      def body(g, carry2):
        pltpu.async_copy(y_hbm.at[src_v.at[pl.ds(g * C2, C2)]], buf,
                         g0).wait()
        pltpu.sync_copy(buf, acc_sh.at[dstp_v.at[pl.ds(g * C2, C2)]],
                        add=True)
        return carry2

      lax.fori_loop(0, SLAB_CH, body, 0)
      return carry

    lax.fori_loop(0, a_c, slab_loop, 0)
    plsc.subcore_barrier()
    pltpu.sync_copy(acc_sh.at[pl.ds(base, RPT)], out_hbm.at[c, s])

  agg_kernel = pl.kernel(
      agg_body,
      out_type=jax.ShapeDtypeStruct((NC, NS, RPT, D), jnp.float32),
      mesh=_mesh,
      scratch_types=[
          pltpu.VMEM((SLAB_E,), jnp.int32),
          pltpu.VMEM((SLAB_E,), jnp.int32),
          pltpu.VMEM((C2, D), jnp.float32),
          pltpu.VMEM_SHARED((NP, D), jnp.float32),
          pltpu.SemaphoreType.DMA,
      ],
  )
  return deg_kernel, agg_kernel


R = 1000


def _tc_prep_body(degp_ref, x_ref, dis_ref, y_ref):
  p = degp_ref[...]
  deg = jnp.sum(p, axis=0)
  dis = jnp.where(deg > 0, lax.rsqrt(jnp.maximum(deg, 1e-12)), 0.0)
  disb = jnp.broadcast_to(dis, (R, D))
  dis_ref[...] = disb
  y_ref[...] = disb * x_ref[...]


def _tc_prep(degp, x):
  return pl.pallas_call(
      _tc_prep_body,
      grid=(N // R,),
      in_specs=[
          pl.BlockSpec((NW, R, 1), lambda i: (0, i, 0)),
          pl.BlockSpec((R, D), lambda i: (i, 0)),
      ],
      out_specs=[
          pl.BlockSpec((R, D), lambda i: (i, 0)),
          pl.BlockSpec((R, D), lambda i: (i, 0)),
      ],
      out_shape=[
          jax.ShapeDtypeStruct((N, D), jnp.float32),
          jax.ShapeDtypeStruct((N, D), jnp.float32),
      ],
  )(degp, x)


def _tc_layer_a_body(x_ref, s0_ref, s1_ref, dis_ref, w0_ref, w1_ref, ba_ref,
                     g_ref, b_ref, h2_ref, y2_ref):
  x = x_ref[...]
  dis = dis_ref[...]
  tx1 = -dis * (s0_ref[...] + s1_ref[...])
  h = (jnp.dot(x, w0_ref[...], preferred_element_type=jnp.float32)
       + jnp.dot(tx1, w1_ref[...], preferred_element_type=jnp.float32)
       + ba_ref[...])
  mu = jnp.mean(h, axis=-1, keepdims=True)
  var = jnp.mean((h - mu) * (h - mu), axis=-1, keepdims=True)
  hn = (h - mu) / jnp.sqrt(var + 1e-5) * g_ref[...] + b_ref[...]
  h2 = jnp.where(hn >= 0, hn, 0.01 * hn)
  h2_ref[...] = h2
  y2_ref[...] = dis * h2


def _tc_layer_a(x, s0, s1, dis, w0, w1, ba, gamma, beta):
  row = lambda i: (i, 0)
  full = lambda i: (0, 0)
  return pl.pallas_call(
      _tc_layer_a_body,
      grid=(N // R,),
      in_specs=[
          pl.BlockSpec((R, D), row),
          pl.BlockSpec((R, D), row),
          pl.BlockSpec((R, D), row),
          pl.BlockSpec((R, D), row),
          pl.BlockSpec((D, D), full),
          pl.BlockSpec((D, D), full),
          pl.BlockSpec((1, D), full),
          pl.BlockSpec((1, D), full),
          pl.BlockSpec((1, D), full),
      ],
      out_specs=[
          pl.BlockSpec((R, D), row),
          pl.BlockSpec((R, D), row),
      ],
      out_shape=[
          jax.ShapeDtypeStruct((N, D), jnp.float32),
          jax.ShapeDtypeStruct((N, D), jnp.float32),
      ],
  )(x, s0, s1, dis, w0, w1, ba.reshape(1, D), gamma.reshape(1, D),
    beta.reshape(1, D))


def _tc_layer_b_body(h_ref, s0_ref, s1_ref, dis_ref, w0_ref, w1_ref, bb_ref,
                     out_ref):
  h = h_ref[...]
  tx1 = -dis_ref[...] * (s0_ref[...] + s1_ref[...])
  out_ref[...] = (jnp.dot(h, w0_ref[...], preferred_element_type=jnp.float32)
                  + jnp.dot(tx1, w1_ref[...], preferred_element_type=jnp.float32)
                  + bb_ref[...])


def _tc_layer_b(h, s0, s1, dis, w0, w1, bb):
  row = lambda i: (i, 0)
  full = lambda i: (0, 0)
  return pl.pallas_call(
      _tc_layer_b_body,
      grid=(N // R,),
      in_specs=[
          pl.BlockSpec((R, D), row),
          pl.BlockSpec((R, D), row),
          pl.BlockSpec((R, D), row),
          pl.BlockSpec((R, D), row),
          pl.BlockSpec((D, D), full),
          pl.BlockSpec((D, D), full),
          pl.BlockSpec((1, D), full),
      ],
      out_specs=pl.BlockSpec((R, D), row),
      out_shape=jax.ShapeDtypeStruct((N, D), jnp.float32),
  )(h, s0, s1, dis, w0, w1, bb.reshape(1, D))


def kernel(x, edge_index, W0a, W1a, ba, gamma, beta, W0b, W1b, bb):
  src = edge_index[0]
  dst = edge_index[1]
  pad = E_PAD - E
  src_p = jnp.concatenate([src, jnp.zeros((pad,), jnp.int32)]).reshape(
      SLABS, SLAB_E)
  dst_p = jnp.concatenate([dst, jnp.full((pad,), DUMMY, jnp.int32)]).reshape(
      SLABS, SLAB_E)
  src_deg = jnp.concatenate([src, jnp.full((pad,), DUMMY, jnp.int32)]).reshape(
      NW, NG * C2)
  _sc_deg, _sc_agg = _make_sc_kernels()

  degp = _sc_deg(src_deg).reshape(NW, NP)[:, :N].reshape(NW, N, 1)
  dis, y = _tc_prep(degp, x)

  sp = _sc_agg(y, src_p, dst_p).reshape(NC, NP, D)
  h2, y2 = _tc_layer_a(x, sp[0, :N], sp[1, :N], dis, W0a, W1a, ba, gamma,
                       beta)

  sp2 = _sc_agg(y2, src_p, dst_p).reshape(NC, NP, D)
  return _tc_layer_b(h2, sp2[0, :N], sp2[1, :N], dis, W0b, W1b, bb)


# speedup vs baseline: 1.1280x; 1.0003x over previous
import functools

import jax
import jax.numpy as jnp
from jax import lax
from jax.experimental import pallas as pl
from jax.experimental.pallas import tpu as pltpu
from jax.experimental.pallas import tpu_sc as plsc

N = 10000
E = 320000
D = 128

NC = 2
NS = 16
NW = NC * NS

C2 = 224
NG = -(-E // (NW * C2))
E_PAD = NW * NG * C2
SLAB_CH = 9
SLAB_E = SLAB_CH * C2
A0 = 6
A1 = 4
SLABS = NS * (A0 + A1)
RPT = -(-(N + 1) // (NS * 8)) * 8
NP = NS * RPT
DUMMY = N


def _zero_buf(buf, rows, cols):
  zeros = jnp.zeros((16,), jnp.float32)

  def body(r, carry):
    for k in range(cols // 16):
      buf[r, pl.ds(k * 16, 16)] = zeros
    return carry

  lax.fori_loop(0, rows, body, 0)


@functools.lru_cache(maxsize=None)
def _make_sc_kernels():
  _mesh = plsc.VectorSubcoreMesh(core_axis_name="c", subcore_axis_name="s")

  E_W = NG * C2

  def deg_body(src_hbm, out_hbm, src_v, histo):
    c = lax.axis_index("c")
    s = lax.axis_index("s")
    w = c * NS + s
    pltpu.sync_copy(src_hbm.at[w], src_v)
    zeros = jnp.zeros((16,), jnp.float32)

    def zrow(r, carry):
      histo[pl.ds(r * 16, 16)] = zeros
      return carry

    lax.fori_loop(0, NP // 16, zrow, 0)
    ones = jnp.full((16,), 1.0, jnp.float32)

    def body(j, carry):
      idx = src_v[pl.ds(j * 16, 16)]
      plsc.addupdate_scatter(histo, [idx], ones)
      return carry

    lax.fori_loop(0, E_W // 16, body, 0)
    pltpu.sync_copy(histo, out_hbm.at[c, s])

  deg_kernel = pl.kernel(
      deg_body,
      out_type=jax.ShapeDtypeStruct((NC, NS, NP), jnp.float32),
      mesh=_mesh,
      compiler_params=pltpu.CompilerParams(needs_layout_passes=False),
      scratch_types=[
          pltpu.VMEM((E_W,), jnp.int32),
          pltpu.VMEM((NP,), jnp.float32),
      ],
  )

  def agg_body(y_hbm, src_hbm, dstp_hbm, out_hbm, src_v, dstp_v,
               buf, acc_sh, g0):
    c = lax.axis_index("c")
    s = lax.axis_index("s")
    _zero_buf(buf, C2, D)
    base = s * RPT
    nfull = RPT // 128
    for j in range(nfull):
      pltpu.sync_copy(buf.at[pl.ds(0, 128)],
                      acc_sh.at[pl.ds(base + j * 128, 128)])
    rem = RPT - nfull * 128
    if rem:
      pltpu.sync_copy(buf.at[pl.ds(0, rem)],
                      acc_sh.at[pl.ds(base + nfull * 128, rem)])
    plsc.subcore_barrier()

    base_c = c * (NS * A0)
    a_c = jnp.where(c == 0, A0, A1)

    def slab_loop(k, carry):
      sid = base_c + k * NS + s
      pltpu.sync_copy(src_hbm.at[sid], src_v)
      pltpu.sync_copy(dstp_hbm.at[sid], dstp_v)

      def body(g, carry2):
        pltpu.async_copy(y_hbm.at[src_v.at[pl.ds(g * C2, C2)]], buf,
                         g0).wait()
        pltpu.sync_copy(buf, acc_sh.at[dstp_v.at[pl.ds(g * C2, C2)]],
                        add=True)
        return carry2

      lax.fori_loop(0, SLAB_CH, body, 0)
      return carry

    lax.fori_loop(0, a_c, slab_loop, 0)
    plsc.subcore_barrier()
    pltpu.sync_copy(acc_sh.at[pl.ds(base, RPT)], out_hbm.at[c, s])

  agg_kernel = pl.kernel(
      agg_body,
      out_type=jax.ShapeDtypeStruct((NC, NS, RPT, D), jnp.float32),
      mesh=_mesh,
      scratch_types=[
          pltpu.VMEM((SLAB_E,), jnp.int32),
          pltpu.VMEM((SLAB_E,), jnp.int32),
          pltpu.VMEM((C2, D), jnp.float32),
          pltpu.VMEM_SHARED((NP, D), jnp.float32),
          pltpu.SemaphoreType.DMA,
      ],
  )
  return deg_kernel, agg_kernel


R = 1000


def _tc_prep_body(degp_ref, x_ref, dis_ref, y_ref):
  p = degp_ref[...]
  deg = jnp.sum(p, axis=0)
  dis = jnp.where(deg > 0, lax.rsqrt(jnp.maximum(deg, 1e-12)), 0.0)
  disb = jnp.broadcast_to(dis, (R, D))
  dis_ref[...] = disb
  y_ref[...] = disb * x_ref[...]


def _tc_prep(degp, x):
  return pl.pallas_call(
      _tc_prep_body,
      grid=(N // R,),
      in_specs=[
          pl.BlockSpec((NW, R, 1), lambda i: (0, i, 0)),
          pl.BlockSpec((R, D), lambda i: (i, 0)),
      ],
      out_specs=[
          pl.BlockSpec((R, D), lambda i: (i, 0)),
          pl.BlockSpec((R, D), lambda i: (i, 0)),
      ],
      out_shape=[
          jax.ShapeDtypeStruct((N, D), jnp.float32),
          jax.ShapeDtypeStruct((N, D), jnp.float32),
      ],
  )(degp, x)


def _tc_layer_a_body(x_ref, s0_ref, s1_ref, dis_ref, w0_ref, w1_ref, ba_ref,
                     g_ref, b_ref, h2_ref, y2_ref):
  x = x_ref[...]
  dis = dis_ref[...]
  tx1 = -dis * (s0_ref[...] + s1_ref[...])
  h = (jnp.dot(x, w0_ref[...], preferred_element_type=jnp.float32)
       + jnp.dot(tx1, w1_ref[...], preferred_element_type=jnp.float32)
       + ba_ref[...])
  mu = jnp.mean(h, axis=-1, keepdims=True)
  var = jnp.mean((h - mu) * (h - mu), axis=-1, keepdims=True)
  hn = (h - mu) / jnp.sqrt(var + 1e-5) * g_ref[...] + b_ref[...]
  h2 = jnp.where(hn >= 0, hn, 0.01 * hn)
  h2_ref[...] = h2
  y2_ref[...] = dis * h2


def _tc_layer_a(x, s0, s1, dis, w0, w1, ba, gamma, beta):
  row = lambda i: (i, 0)
  full = lambda i: (0, 0)
  return pl.pallas_call(
      _tc_layer_a_body,
      grid=(N // R,),
      in_specs=[
          pl.BlockSpec((R, D), row),
          pl.BlockSpec((R, D), row),
          pl.BlockSpec((R, D), row),
          pl.BlockSpec((R, D), row),
          pl.BlockSpec((D, D), full),
          pl.BlockSpec((D, D), full),
          pl.BlockSpec((1, D), full),
          pl.BlockSpec((1, D), full),
          pl.BlockSpec((1, D), full),
      ],
      out_specs=[
          pl.BlockSpec((R, D), row),
          pl.BlockSpec((R, D), row),
      ],
      out_shape=[
          jax.ShapeDtypeStruct((N, D), jnp.float32),
          jax.ShapeDtypeStruct((N, D), jnp.float32),
      ],
  )(x, s0, s1, dis, w0, w1, ba.reshape(1, D), gamma.reshape(1, D),
    beta.reshape(1, D))


def _tc_layer_b_body(h_ref, s0_ref, s1_ref, dis_ref, w0_ref, w1_ref, bb_ref,
                     out_ref):
  h = h_ref[...]
  tx1 = -dis_ref[...] * (s0_ref[...] + s1_ref[...])
  out_ref[...] = (jnp.dot(h, w0_ref[...], preferred_element_type=jnp.float32)
                  + jnp.dot(tx1, w1_ref[...], preferred_element_type=jnp.float32)
                  + bb_ref[...])


def _tc_layer_b(h, s0, s1, dis, w0, w1, bb):
  row = lambda i: (i, 0)
  full = lambda i: (0, 0)
  return pl.pallas_call(
      _tc_layer_b_body,
      grid=(N // R,),
      in_specs=[
          pl.BlockSpec((R, D), row),
          pl.BlockSpec((R, D), row),
          pl.BlockSpec((R, D), row),
          pl.BlockSpec((R, D), row),
          pl.BlockSpec((D, D), full),
          pl.BlockSpec((D, D), full),
          pl.BlockSpec((1, D), full),
      ],
      out_specs=pl.BlockSpec((R, D), row),
      out_shape=jax.ShapeDtypeStruct((N, D), jnp.float32),
  )(h, s0, s1, dis, w0, w1, bb.reshape(1, D))


def kernel(x, edge_index, W0a, W1a, ba, gamma, beta, W0b, W1b, bb):
  src = edge_index[0]
  dst = edge_index[1]
  pad = E_PAD - E
  src_p = jnp.concatenate([src, jnp.zeros((pad,), jnp.int32)]).reshape(
      SLABS, SLAB_E)
  dst_p = jnp.concatenate([dst, jnp.full((pad,), DUMMY, jnp.int32)]).reshape(
      SLABS, SLAB_E)
  src_deg = jnp.concatenate([src, jnp.full((pad,), DUMMY, jnp.int32)]).reshape(
      NW, NG * C2)
  _sc_deg, _sc_agg = _make_sc_kernels()

  degp = _sc_deg(src_deg).reshape(NW, NP)[:, :N].reshape(NW, N, 1)
  dis, y = _tc_prep(degp, x)

  sp = _sc_agg(y, src_p, dst_p).reshape(NC, NP, D)
  h2, y2 = _tc_layer_a(x, sp[0, :N], sp[1, :N], dis, W0a, W1a, ba, gamma,
                       beta)

  sp2 = _sc_agg(y2, src_p, dst_p).reshape(NC, NP, D)
  return _tc_layer_b(h2, sp2[0, :N], sp2[1, :N], dis, W0b, W1b, bb)
